# baseline (speedup 1.0000x reference)
import functools

import jax
import jax.numpy as jnp
from jax import lax
from jax.experimental import pallas as pl
from jax.experimental.pallas import tpu as pltpu
from jax.experimental.pallas import tpu_sc as plsc

D = 128
NC, NS = 2, 16
NW = NC * NS
E = 16
CC = 40
SC_E = CC * E

_SIZES = (1024, 512, 1024, 2048, 2)
_OFFS = (0, 1024, 1536, 2560, 4608)
_STREAM_OFFS = (0, 1024, 1536, 4608)
_ACT_OFF = 2560
_ZERO_ROW = 4610
_ROWS = 4616


@functools.cache
def _make_gather_sum(N: int):
    per_w = N // NW
    assert per_w * NW == N and per_w % SC_E == 0
    n_super = per_w // SC_E
    mesh = plsc.VectorSubcoreMesh(core_axis_name="c", subcore_axis_name="s")

    @functools.partial(
        pl.kernel,
        mesh=mesh,
        out_type=jax.ShapeDtypeStruct((N, D), jnp.float32),
        scratch_types=[
            pltpu.VMEM((4, SC_E), jnp.int32),
            pltpu.VMEM((4 * SC_E,), jnp.int32),
            pltpu.VMEM((4 * SC_E,), jnp.int32),
            pltpu.VMEM((8, CC, E), jnp.int32),
            pltpu.VMEM((2, 8, E, D), jnp.float32),
            pltpu.VMEM((2, E, D), jnp.float32),
            pltpu.VMEM_SHARED((_ROWS, D), jnp.float32),
            pltpu.SemaphoreType.DMA,
            pltpu.SemaphoreType.DMA,
            pltpu.SemaphoreType.DMA,
            pltpu.SemaphoreType.DMA,
        ],
    )
    def gather_sum(sp_hbm, ab_hbm, it_hbm, sd_hbm, mv_hbm, spx_hbm,
                   t0_hbm, t1_hbm, t2_hbm, t3_hbm, t4_hbm, out_hbm,
                   raw_v, mvf_v, spx_v, gidx_v, buf_v, acc_v, sp_table,
                   gsem0, gsem1, osem0, osem1):
        wid = lax.axis_index("s") * NC + lax.axis_index("c")
        w_base = wid * per_w
        sid = lax.axis_index("s")
        gsem = (gsem0, gsem1)
        osem = (osem0, osem1)

        @pl.when(sid == 0)
        def _stage():
            for tab, off, size in zip((t0_hbm, t1_hbm, t2_hbm, t3_hbm, t4_hbm),
                                      _OFFS, _SIZES):
                pltpu.sync_copy(tab, sp_table.at[pl.ds(off, size)])
            for j in range(D // 16):
                acc_v[0, 0, pl.ds(j * 16, 16)] = jnp.zeros((16,), jnp.float32)
            pltpu.sync_copy(acc_v.at[0, 0], sp_table.at[_ZERO_ROW])

        plsc.subcore_barrier()

        for b in (0, 1):
            pltpu.async_copy(out_hbm.at[pl.ds(w_base, E)], acc_v.at[b], osem[b])

        def fire(c, b):
            for k in range(8):
                pltpu.async_copy(sp_table.at[gidx_v.at[k, c]],
                                 buf_v.at[b, k], gsem[b])

        def consume(c, sbase, b):
            for k in range(8):
                pltpu.make_async_copy(sp_table.at[gidx_v.at[k, 0]],
                                      buf_v.at[b, k], gsem[b]).wait()
            pltpu.make_async_copy(acc_v.at[b], out_hbm.at[pl.ds(w_base, E)],
                                  osem[b]).wait()

            def grp_body(g, c2):
                se = 4 + g
                for e4 in range(4):
                    e = g * 4 + e4
                    lb = 4 * e4
                    for j in range(D // 16):
                        slj = pl.ds(j * 16, 16)
                        a = buf_v[b, 0, e, slj]
                        for k in (1, 2, 3):
                            a = a + buf_v[b, k, e, slj]
                        for q in range(4):
                            a = a + buf_v[b, se, lb + q, slj]
                        acc_v[b, e, slj] = a
                return c2

            lax.fori_loop(0, E // 4, grp_body, 0)
            pltpu.async_copy(acc_v.at[b], out_hbm.at[pl.ds(sbase + c * E, E)],
                             osem[b])

        def super_body(s, carry):
            sbase = w_base + s * SC_E
            for k, ref in enumerate((sp_hbm, ab_hbm, it_hbm, sd_hbm)):
                pltpu.sync_copy(ref.at[pl.ds(sbase, SC_E)], raw_v.at[k])
            pltpu.sync_copy(mv_hbm.at[pl.ds(sbase * 4, SC_E * 4)], mvf_v)
            pltpu.sync_copy(spx_hbm.at[pl.ds(sbase * 4, SC_E * 4)], spx_v)

            def munge_body(cc, c2):
                sl = pl.ds(cc * E, 16)
                sp = raw_v[0, sl]
                mask = sp != 0
                gidx_v[0, cc, pl.ds(0, 16)] = jnp.where(mask, sp, _ZERO_ROW)
                for k in (1, 2, 3):
                    v = raw_v[k, sl] + _STREAM_OFFS[k]
                    gidx_v[k, cc, pl.ds(0, 16)] = jnp.where(mask, v, _ZERO_ROW)
                for m in range(4):
                    slx = pl.ds(cc * 64 + m * 16, 16)
                    maskx = spx_v[slx] != 0
                    gidx_v[4 + m, cc, pl.ds(0, 16)] = jnp.where(
                        maskx, mvf_v[slx] + _ACT_OFF, _ZERO_ROW)
                return c2

            lax.fori_loop(0, CC, munge_body, 0)

            fire(0, 0)

            def pair_body(t, c2):
                fire(2 * t + 1, 1)
                consume(2 * t, sbase, 0)
                fire(2 * t + 2, 0)
                consume(2 * t + 1, sbase, 1)
                return c2

            lax.fori_loop(0, CC // 2 - 1, pair_body, 0)
            fire(CC - 1, 1)
            consume(CC - 2, sbase, 0)
            consume(CC - 1, sbase, 1)
            return carry

        lax.fori_loop(0, n_super, super_body, 0)

        for b in (0, 1):
            pltpu.make_async_copy(acc_v.at[b], out_hbm.at[pl.ds(w_base, E)],
                                  osem[b]).wait()

    return gather_sum


def kernel(species_idx, ability_idx, item_idx, side_idx, moveset_idx,
           species_table, abilities_table, items_table, actions_table, side_table):
    B, T = species_idx.shape
    N = B * T
    sp_n = species_idx.reshape(N).astype(jnp.int32)
    out = _make_gather_sum(N)(
        sp_n,
        ability_idx.reshape(N).astype(jnp.int32),
        item_idx.reshape(N).astype(jnp.int32),
        side_idx.reshape(N).astype(jnp.int32),
        moveset_idx.reshape(N * 4).astype(jnp.int32),
        jnp.broadcast_to(sp_n[:, None], (N, 4)).reshape(N * 4),
        species_table, abilities_table, items_table, actions_table, side_table,
    )
    return out.reshape(B, T, D)


# speedup vs baseline: 2.3087x; 2.3087x over previous
import functools

import jax
import jax.numpy as jnp
from jax import lax
from jax.experimental import pallas as pl
from jax.experimental.pallas import tpu as pltpu
from jax.experimental.pallas import tpu_sc as plsc

D = 128
NC, NS = 2, 16
NW = NC * NS
E = 16
CC = 40
SC_E = CC * E

_OFFS = (0, 1024, 1536, 4608, 2560, 2560, 2560, 2560)
_ZERO_ROW = 4610
_ROWS = 4616
_REP = 32


@functools.cache
def _make_gather_sum(N: int):
    per_w = N // NW
    assert per_w * NW == N and per_w % SC_E == 0
    n_super = per_w // SC_E
    mesh = plsc.VectorSubcoreMesh(core_axis_name="c", subcore_axis_name="s")

    @functools.partial(
        pl.kernel,
        mesh=mesh,
        out_type=jax.ShapeDtypeStruct((N, D), jnp.float32),
        scratch_types=[
            pltpu.VMEM((8, SC_E), jnp.int32),
            pltpu.VMEM((8, CC, E), jnp.int32),
            pltpu.VMEM((2, 8, E, D), jnp.float32),
            pltpu.VMEM((2, E, D), jnp.float32),
            pltpu.VMEM_SHARED((_ROWS, D), jnp.float32),
            pltpu.SemaphoreType.DMA,
            pltpu.SemaphoreType.DMA,
            pltpu.SemaphoreType.DMA,
            pltpu.SemaphoreType.DMA,
        ],
    )
    def gather_sum(table_hbm, idx_hbm, out_hbm, raw_v, gidx_v, buf_v, acc_v,
                   sp_table, gsem0, gsem1, osem0, osem1):
        wid = lax.axis_index("s") * NC + lax.axis_index("c")
        w_base = wid * per_w
        sid = lax.axis_index("s")

        @pl.when(sid == 0)
        def _stage():
            pltpu.sync_copy(table_hbm.at[pl.ds(0, _ROWS)], sp_table)

        plsc.subcore_barrier()
        gsem = (gsem0, gsem1)
        osem = (osem0, osem1)

        for b in (0, 1):
            pltpu.async_copy(out_hbm.at[pl.ds(w_base, E)], acc_v.at[b], osem[b])

        def fire(c, b):
            for k in range(8):
                pltpu.async_copy(sp_table.at[gidx_v.at[k, c]],
                                 buf_v.at[b, k], gsem[b])

        def consume(c, sbase, b):
            for k in range(8):
                pltpu.make_async_copy(sp_table.at[gidx_v.at[k, 0]],
                                      buf_v.at[b, k], gsem[b]).wait()
            pltpu.make_async_copy(acc_v.at[b], out_hbm.at[pl.ds(w_base, E)],
                                  osem[b]).wait()

            def row_body(e, c2):
                for j in range(D // 16):
                    slj = pl.ds(j * 16, 16)
                    a = buf_v[b, 0, e, slj]
                    for k in range(1, 8):
                        a = a + buf_v[b, k, e, slj]
                    acc_v[b, e, slj] = a
                return c2

            lax.fori_loop(0, E, row_body, 0)
            pltpu.async_copy(acc_v.at[b], out_hbm.at[pl.ds(sbase + c * E, E)],
                             osem[b])

        def super_body(s, carry):
            sbase = w_base + s * SC_E
            for k in range(8):
                pltpu.sync_copy(idx_hbm.at[k, pl.ds(sbase, SC_E)], raw_v.at[k])

            def munge_body(cc, c2):
                for half in range(E // 16):
                    sl = pl.ds(cc * E + half * 16, 16)
                    sp = raw_v[0, sl]
                    mask = sp != 0
                    for k in range(8):
                        v = sp if k == 0 else raw_v[k, sl] + _OFFS[k]
                        gidx_v[k, cc, pl.ds(half * 16, 16)] = jnp.where(
                            mask, v, _ZERO_ROW)
                return c2

            lax.fori_loop(0, CC, munge_body, 0)

            fire(0, 0)

            def pair_body(t, c2):
                fire(2 * t + 1, 1)
                consume(2 * t, sbase, 0)
                fire(2 * t + 2, 0)
                consume(2 * t + 1, sbase, 1)
                return c2

            lax.fori_loop(0, CC // 2 - 1, pair_body, 0)
            fire(CC - 1, 1)
            consume(CC - 2, sbase, 0)
            consume(CC - 1, sbase, 1)
            return carry

        lax.fori_loop(0, n_super, super_body, 0)

        for b in (0, 1):
            pltpu.make_async_copy(acc_v.at[b], out_hbm.at[pl.ds(w_base, E)],
                                  osem[b]).wait()

    return gather_sum


def kernel(species_idx, ability_idx, item_idx, side_idx, moveset_idx,
           species_table, abilities_table, items_table, actions_table, side_table):
    B, T = species_idx.shape
    N = B * T
    mv = moveset_idx.reshape(N, 4).astype(jnp.int32)
    idx_stack = jnp.stack([
        species_idx.reshape(N).astype(jnp.int32),
        ability_idx.reshape(N).astype(jnp.int32),
        item_idx.reshape(N).astype(jnp.int32),
        side_idx.reshape(N).astype(jnp.int32),
        mv[:, 0], mv[:, 1], mv[:, 2], mv[:, 3],
    ])
    pad = jnp.zeros((_ROWS - 4610, D), jnp.float32)
    table = jnp.concatenate(
        [species_table, abilities_table, items_table, actions_table, side_table, pad],
        axis=0)
    out = _make_gather_sum(N)(table, idx_stack)
    return out.reshape(B, T, D)
